# Initial kernel scaffold; baseline (speedup 1.0000x reference)
import functools

import jax
import jax.numpy as jnp
from jax import lax
from jax.experimental import pallas as pl
from jax.experimental.pallas import tpu as pltpu
from jax.experimental.pallas import tpu_sc as plsc

K = 5
B, S, D = 4, 8192, 1024
NC, NS, L = 2, 16, 16
NW = NC * NS
COLS = 32
NCG = COLS // L
TASKS = (B * D) // COLS
TASKS_PER_W = TASKS // NW
S_CHUNK = 512
N_CHUNKS = S // S_CHUNK
UNROLL = 8
NEG_INF = float("-inf")


def _insert(ms, v):
    out = []
    carry = v
    for m in ms:
        out.append(jnp.maximum(m, carry))
        carry = jnp.minimum(m, carry)
    return out


def _kmax_body(x_hbm, out_hbm, buf, ostage, sem):
    cid = lax.axis_index("c")
    sid = lax.axis_index("s")
    wid = sid * NC + cid

    for ti in range(TASKS_PER_W):
        t = wid * TASKS_PER_W + ti
        b = t // (D // COLS)
        d0 = (t % (D // COLS)) * COLS

        state = tuple(
            jnp.full((L,), NEG_INF, jnp.float32) for _ in range(NCG * K)
        )

        def chunk_body(c, state, b=b, d0=d0):
            s0 = c * S_CHUNK
            pltpu.async_copy(
                x_hbm.at[b, pl.ds(s0, S_CHUNK), pl.ds(d0, COLS)], buf, sem
            ).wait()

            def rows_body(i, state):
                st = list(state)
                for u in range(UNROLL):
                    r = i * UNROLL + u
                    for cg in range(NCG):
                        v = buf[r, pl.ds(cg * L, L)]
                        st[cg * K:(cg + 1) * K] = _insert(
                            st[cg * K:(cg + 1) * K], v
                        )
                return tuple(st)

            return lax.fori_loop(0, S_CHUNK // UNROLL, rows_body, state)

        state = lax.fori_loop(0, N_CHUNKS, chunk_body, state)

        lane = jnp.arange(L, dtype=jnp.int32)
        for cg in range(NCG):
            for j in range(K):
                idx = (lane + cg * L) * K + j
                plsc.store_scatter(ostage, [idx], state[cg * K + j])
        pltpu.sync_copy(ostage, out_hbm.at[b, pl.ds(d0 * K, COLS * K)])


@jax.jit
def kernel(inputs):
    mesh = plsc.VectorSubcoreMesh(
        core_axis_name="c", subcore_axis_name="s", num_cores=NC,
        num_subcores=NS,
    )
    kfn = pl.kernel(
        _kmax_body,
        out_type=jax.ShapeDtypeStruct((B, D * K), jnp.float32),
        mesh=mesh,
        scratch_types=[
            pltpu.VMEM((S_CHUNK, COLS), jnp.float32),
            pltpu.VMEM((COLS * K,), jnp.float32),
            pltpu.SemaphoreType.DMA,
        ],
    )
    return kfn(inputs)


# speedup vs baseline: 28.8738x; 28.8738x over previous
import functools

import jax
import jax.numpy as jnp
from jax import lax
from jax.experimental import pallas as pl
from jax.experimental.pallas import tpu as pltpu
from jax.experimental.pallas import tpu_sc as plsc

K = 5
B, S, D = 4, 8192, 1024
NC, NS, L = 2, 16, 16
NW = NC * NS
COLS = 32
NCG = COLS // L
TASKS = (B * D) // COLS
TASKS_PER_W = TASKS // NW
S_CHUNK = 512
N_CHUNKS = S // S_CHUNK
UNROLL = 8
NEG_INF = float("-inf")


def _insert(ms, v):
    out = []
    carry = v
    for m in ms:
        out.append(jnp.maximum(m, carry))
        carry = jnp.minimum(m, carry)
    return out


def _kmax_body(x_hbm, out_hbm, buf, ostage, sem):
    cid = lax.axis_index("c")
    sid = lax.axis_index("s")
    wid = sid * NC + cid

    for ti in range(TASKS_PER_W):
        t = wid * TASKS_PER_W + ti
        b = t // (D // COLS)
        d0 = (t % (D // COLS)) * COLS

        state = tuple(
            jnp.full((L,), NEG_INF, jnp.float32) for _ in range(NCG * K)
        )

        def chunk_body(c, state, b=b, d0=d0):
            s0 = c * S_CHUNK
            pltpu.async_copy(
                x_hbm.at[b, pl.ds(s0, S_CHUNK), pl.ds(d0, COLS)], buf, sem
            ).wait()

            def rows_body(i, state):
                st = list(state)
                for u in range(UNROLL):
                    r = i * UNROLL + u
                    for cg in range(NCG):
                        v = buf[r, pl.ds(cg * L, L)]
                        st[cg * K:(cg + 1) * K] = _insert(
                            st[cg * K:(cg + 1) * K], v
                        )
                return tuple(st)

            return lax.fori_loop(0, S_CHUNK // UNROLL, rows_body, state)

        state = lax.fori_loop(0, N_CHUNKS, chunk_body, state)

        lane = jnp.arange(L, dtype=jnp.int32)
        for cg in range(NCG):
            for j in range(K):
                idx = (lane + cg * L) * K + j
                plsc.store_scatter(ostage, [idx], state[cg * K + j])
        pltpu.sync_copy(ostage, out_hbm.at[b, pl.ds(d0 * K, COLS * K)])


@jax.jit
def kernel(inputs):
    mesh = plsc.VectorSubcoreMesh(
        core_axis_name="c", subcore_axis_name="s", num_cores=NC,
        num_subcores=NS,
    )
    kfn = pl.kernel(
        _kmax_body,
        out_type=jax.ShapeDtypeStruct((B, D * K), jnp.float32),
        mesh=mesh,
        scratch_types=[
            pltpu.VMEM((S_CHUNK, COLS), jnp.float32),
            pltpu.VMEM((COLS * K,), jnp.float32),
            pltpu.SemaphoreType.DMA,
        ],
        compiler_params=pltpu.CompilerParams(
            use_tc_tiling_on_sc=False, needs_layout_passes=False
        ),
    )
    return kfn(inputs)


# speedup vs baseline: 41.7785x; 1.4469x over previous
import jax
import jax.numpy as jnp
from jax import lax
from jax.experimental import pallas as pl
from jax.experimental.pallas import tpu as pltpu
from jax.experimental.pallas import tpu_sc as plsc

K = 5
B, S, D = 4, 8192, 1024
NC, NS, L = 2, 16, 16
NW = NC * NS
STRIPS = (B * D) // L
STRIPS_PER_W = STRIPS // NW
G = 16
BLK = 2048
NGRP = BLK // G
NBLK = S // BLK
NQ = STRIPS_PER_W * NBLK
NEG_INF = float("-inf")


def _insert(ms, v):
    out = []
    carry = v
    for m in ms:
        out.append(jnp.maximum(m, carry))
        carry = jnp.minimum(m, carry)
    return out


def _insert_with_idx(vals, idxs, v, vi):
    nv, ni = [], []
    cv, ci = v, vi
    for m, mi in zip(vals, idxs):
        take = cv > m
        nv.append(jnp.maximum(m, cv))
        ni.append(jnp.where(take, ci, mi))
        cv = jnp.minimum(m, cv)
        ci = jnp.where(take, mi, ci)
    return nv, ni


def _kmax_body(x_hbm, out_hbm, buf, gidbuf, ostage, sem):
    cid = lax.axis_index("c")
    sid = lax.axis_index("s")
    wid = sid * NC + cid
    lane = jnp.arange(L, dtype=jnp.int32)

    def src_slice(qq):
        strip = wid * STRIPS_PER_W + qq // NBLK
        b = strip // (D // L)
        c0 = (strip % (D // L)) * L
        row0 = b * S + (qq % NBLK) * BLK
        return x_hbm.at[pl.ds(row0, BLK), pl.ds(c0, L)]

    pltpu.async_copy(src_slice(0), buf.at[0], sem.at[0])

    def step(q, state):
        par = q % 2
        pltpu.make_async_copy(src_slice(q), buf.at[par], sem.at[par]).wait()

        @pl.when(q + 1 < NQ)
        def _():
            pltpu.async_copy(
                src_slice(q + 1), buf.at[1 - par], sem.at[1 - par]
            )

        def grp(g, carry):
            vals, idxs = carry[:K], carry[K:]
            r0 = g * G
            vs = [buf[par, r0 + r, :] for r in range(G)]
            while len(vs) > 1:
                vs = [
                    jnp.maximum(vs[2 * i], vs[2 * i + 1])
                    for i in range(len(vs) // 2)
                ]
            gmax = vs[0]
            gid = jnp.zeros((L,), jnp.int32) + g
            vals, idxs = _insert_with_idx(vals, idxs, gmax, gid)
            return tuple(vals) + tuple(idxs)

        cinit = tuple(jnp.full((L,), NEG_INF, jnp.float32) for _ in range(K))
        cinit += tuple(jnp.zeros((L,), jnp.int32) for _ in range(K))
        cand = lax.fori_loop(0, NGRP, grp, cinit)
        for j in range(K):
            gidbuf[j, :] = cand[K + j]

        fresh = q % NBLK == 0
        state = tuple(
            jnp.where(fresh, jnp.full((L,), NEG_INF, jnp.float32), m)
            for m in state
        )

        def cand_j(j, state):
            gid = gidbuf[j, :]

            def row_r(r, state):
                v = plsc.load_gather(
                    buf, [jnp.zeros((L,), jnp.int32) + par, gid * G + r, lane]
                )
                return tuple(_insert(list(state), v))

            return lax.fori_loop(0, G, row_r, state)

        state = lax.fori_loop(0, K, cand_j, state)

        @pl.when(q % NBLK == NBLK - 1)
        def _():
            strip = wid * STRIPS_PER_W + q // NBLK
            b = strip // (D // L)
            c0 = (strip % (D // L)) * L
            for j in range(K):
                plsc.store_scatter(ostage, [lane * K + j], state[j])
            pltpu.sync_copy(
                ostage, out_hbm.at[pl.ds(b * D * K + c0 * K, L * K)]
            )

        return state

    state0 = tuple(jnp.full((L,), NEG_INF, jnp.float32) for _ in range(K))
    lax.fori_loop(0, NQ, step, state0)


@jax.jit
def kernel(inputs):
    mesh = plsc.VectorSubcoreMesh(
        core_axis_name="c", subcore_axis_name="s", num_cores=NC,
        num_subcores=NS,
    )
    kfn = pl.kernel(
        _kmax_body,
        out_type=jax.ShapeDtypeStruct((B * D * K,), jnp.float32),
        mesh=mesh,
        scratch_types=[
            pltpu.VMEM((2, BLK, L), jnp.float32),
            pltpu.VMEM((K, L), jnp.int32),
            pltpu.VMEM((L * K,), jnp.float32),
            pltpu.SemaphoreType.DMA((2,)),
        ],
        compiler_params=pltpu.CompilerParams(
            use_tc_tiling_on_sc=False, needs_layout_passes=False
        ),
    )
    out = kfn(inputs.reshape(B * S, D))
    return out.reshape(B, D * K)


# speedup vs baseline: 41.8115x; 1.0008x over previous
import jax
import jax.numpy as jnp
from jax import lax
from jax.experimental import pallas as pl
from jax.experimental.pallas import tpu as pltpu
from jax.experimental.pallas import tpu_sc as plsc

K = 5
B, S, D = 4, 8192, 1024
NC, NS, L = 2, 16, 16
NW = NC * NS
STRIPS = (B * D) // L
STRIPS_PER_W = STRIPS // NW
G = 16
BLK = 2048
NGRP = BLK // G
NBLK = S // BLK
NQ = STRIPS_PER_W * NBLK
NEG_INF = float("-inf")


def _insert(ms, v):
    out = []
    carry = v
    for m in ms:
        out.append(jnp.maximum(m, carry))
        carry = jnp.minimum(m, carry)
    return out


def _insert_with_idx(vals, idxs, v, vi):
    nv, ni = [], []
    cv, ci = v, vi
    for m, mi in zip(vals, idxs):
        take = cv > m
        nv.append(jnp.maximum(m, cv))
        ni.append(jnp.where(take, ci, mi))
        cv = jnp.minimum(m, cv)
        ci = jnp.where(take, mi, ci)
    return nv, ni


def _kmax_body(x_hbm, out_hbm, buf, gidbuf, ostage, sem):
    cid = lax.axis_index("c")
    sid = lax.axis_index("s")
    wid = sid * NC + cid
    lane = jnp.arange(L, dtype=jnp.int32)

    def src_slice(qq):
        strip = wid * STRIPS_PER_W + qq // NBLK
        b = strip // (D // L)
        c0 = (strip % (D // L)) * L
        row0 = b * S + (qq % NBLK) * BLK
        return x_hbm.at[pl.ds(row0, BLK), pl.ds(c0, L)]

    pltpu.async_copy(src_slice(0), buf.at[0], sem.at[0])

    def step(q, state):
        par = q % 2
        pltpu.make_async_copy(src_slice(q), buf.at[par], sem.at[par]).wait()

        @pl.when(q + 1 < NQ)
        def _():
            pltpu.async_copy(
                src_slice(q + 1), buf.at[1 - par], sem.at[1 - par]
            )

        def grp(g, carry):
            vals, idxs = carry[:K], carry[K:]
            r0 = g * G
            vs = [buf[par, r0 + r, :] for r in range(G)]
            while len(vs) > 1:
                vs = [
                    jnp.maximum(vs[2 * i], vs[2 * i + 1])
                    for i in range(len(vs) // 2)
                ]
            gmax = vs[0]
            gid = jnp.zeros((L,), jnp.int32) + g
            vals, idxs = _insert_with_idx(vals, idxs, gmax, gid)
            return tuple(vals) + tuple(idxs)

        cinit = tuple(jnp.full((L,), NEG_INF, jnp.float32) for _ in range(K))
        cinit += tuple(jnp.zeros((L,), jnp.int32) for _ in range(K))
        cand = lax.fori_loop(0, NGRP, grp, cinit, unroll=4)
        for j in range(K):
            gidbuf[j, :] = cand[K + j]

        fresh = q % NBLK == 0
        state = tuple(
            jnp.where(fresh, jnp.full((L,), NEG_INF, jnp.float32), m)
            for m in state
        )

        def cand_j(j, state):
            gid = gidbuf[j, :]
            base = gid * G

            def row_r(r, state):
                sa, sb = state[:K], state[K:]
                pv = jnp.zeros((L,), jnp.int32) + par
                va = plsc.load_gather(buf, [pv, base + r, lane])
                vb = plsc.load_gather(buf, [pv, base + (r + G // 2), lane])
                return tuple(_insert(list(sa), va)) + tuple(
                    _insert(list(sb), vb)
                )

            return lax.fori_loop(0, G // 2, row_r, state)

        state = lax.fori_loop(0, K, cand_j, state)

        @pl.when(q % NBLK == NBLK - 1)
        def _():
            merged = list(state[:K])
            for m in state[K:]:
                merged = _insert(merged, m)
            strip = wid * STRIPS_PER_W + q // NBLK
            b = strip // (D // L)
            c0 = (strip % (D // L)) * L
            for j in range(K):
                plsc.store_scatter(ostage, [lane * K + j], merged[j])
            pltpu.sync_copy(
                ostage, out_hbm.at[pl.ds(b * D * K + c0 * K, L * K)]
            )

        return state

    state0 = tuple(
        jnp.full((L,), NEG_INF, jnp.float32) for _ in range(2 * K)
    )
    lax.fori_loop(0, NQ, step, state0)


@jax.jit
def kernel(inputs):
    mesh = plsc.VectorSubcoreMesh(
        core_axis_name="c", subcore_axis_name="s", num_cores=NC,
        num_subcores=NS,
    )
    kfn = pl.kernel(
        _kmax_body,
        out_type=jax.ShapeDtypeStruct((B * D * K,), jnp.float32),
        mesh=mesh,
        scratch_types=[
            pltpu.VMEM((2, BLK, L), jnp.float32),
            pltpu.VMEM((K, L), jnp.int32),
            pltpu.VMEM((L * K,), jnp.float32),
            pltpu.SemaphoreType.DMA((2,)),
        ],
        compiler_params=pltpu.CompilerParams(
            use_tc_tiling_on_sc=False, needs_layout_passes=False
        ),
    )
    out = kfn(inputs.reshape(B * S, D))
    return out.reshape(B, D * K)
